# baseline (speedup 1.0000x reference)
import functools

import numpy as np
import jax
import jax.numpy as jnp
from jax.experimental import pallas as pl
from jax.experimental.pallas import tpu as pltpu

_B, _H, _L, _F = 2, 12, 2048, 64
_BH = _B * _H
_NQ = int(np.ceil(np.log(_L)) * 5)
_NK = _NQ
_KT = 512
_HIGH = jax.lax.Precision.HIGHEST
_DEF = jax.lax.Precision.DEFAULT

_INTERPRET = False


def _rotl32(x, r):
    return ((x << np.uint32(r)) | (x >> np.uint32(32 - r))).astype(np.uint32)


def _threefry2x32_np(ks0, ks1, x0, x1):
    x0 = x0.astype(np.uint32).copy()
    x1 = x1.astype(np.uint32).copy()
    ks2 = np.uint32(np.uint32(ks0) ^ np.uint32(ks1) ^ np.uint32(0x1BD11BDA))
    rots = ((13, 15, 26, 6), (17, 29, 16, 24))
    x0 += np.uint32(ks0)
    x1 += np.uint32(ks1)
    inject = [(ks1, ks2), (ks2, ks0), (ks0, ks1), (ks1, ks2), (ks2, ks0)]
    for d in range(5):
        for r in rots[d % 2]:
            x0 = (x0 + x1).astype(np.uint32)
            x1 = _rotl32(x1, r)
            x1 = x1 ^ x0
        a, b = inject[d]
        x0 = (x0 + np.uint32(a)).astype(np.uint32)
        x1 = (x1 + np.uint32(b) + np.uint32(d + 1)).astype(np.uint32)
    return x0, x1


def _np_randint_key1(shape, span):
    size = int(np.prod(shape))
    flat = np.arange(size, dtype=np.uint64)
    c1 = (flat >> np.uint64(32)).astype(np.uint32)
    c2 = (flat & np.uint64(0xFFFFFFFF)).astype(np.uint32)
    sb1, sb2 = _threefry2x32_np(np.uint32(0), np.uint32(1),
                                np.zeros(2, np.uint32), np.arange(2, dtype=np.uint32))
    h1, h2 = _threefry2x32_np(sb1[0], sb2[0], c1, c2)
    l1, l2 = _threefry2x32_np(sb1[1], sb2[1], c1, c2)
    higher = (h1 ^ h2).reshape(shape)
    lower = (l1 ^ l2).reshape(shape)
    spanu = np.uint32(span)
    mult = np.uint32((int(2 ** 16) % int(span)) ** 2 % int(span))
    off = ((higher % spanu) * mult + (lower % spanu)) % spanu
    return off.astype(np.int32)


def _build_cnt() -> np.ndarray:
    try:
        cpu = jax.devices("cpu")[0]
        with jax.default_device(cpu):
            idx = np.asarray(jax.random.randint(jax.random.key(1), (_L, _NK), 0, _L))
    except Exception:
        idx = _np_randint_key1((_L, _NK), _L)
    cnt = np.zeros((_L, _L), np.float32)
    np.add.at(cnt, (np.arange(_L)[:, None], idx), 1.0)
    import ml_dtypes
    return cnt.T.astype(ml_dtypes.bfloat16)


_CNT_T = _build_cnt()


def _a_body(q_ref, k_ref, cnt_ref, m_ref, smax, ssum):
    kt = pl.program_id(1)
    st = jax.lax.dot_general(k_ref[0, 0], q_ref[0, 0], (((1,), (1,)), ((), ())),
                             preferred_element_type=jnp.float32,
                             precision=_HIGH)
    cf = cnt_ref[...].astype(jnp.float32)
    pmax = jnp.max(jnp.where(cf > 0.0, st, -jnp.inf), axis=0, keepdims=True)
    psum = jnp.sum(cf * st, axis=0, keepdims=True)

    @pl.when(kt == 0)
    def _():
        smax[...] = pmax
        ssum[...] = psum

    @pl.when(kt > 0)
    def _():
        smax[...] = jnp.maximum(smax[...], pmax)
        ssum[...] = ssum[...] + psum

    @pl.when(kt == (_L // _KT) - 1)
    def _():
        m_ref[0] = smax[...] - ssum[...] * (1.0 / _L)


def _topk_body(m_ref, idx_ref):
    mv = m_ref[:, 0, :]
    lidx = jax.lax.broadcasted_iota(jnp.int32, (_BH, _L), 1)
    lane = jax.lax.broadcasted_iota(jnp.int32, (_BH, 128), 1)
    acc = jnp.zeros((_BH, 128), jnp.int32)
    for i in range(_NQ):
        mx = jnp.max(mv, axis=1, keepdims=True)
        il = jnp.min(jnp.where(mv == mx, lidx, _L), axis=1, keepdims=True)
        acc = acc + jnp.where(lane == i, jnp.broadcast_to(il, (_BH, 128)), 0)
        mv = jnp.where(lidx == il, -jnp.inf, mv)
    idx_ref[...] = acc


def _attn_body(sref, q_ref, k_ref, v_ref, vn_ref, qred):
    bh = pl.program_id(0)
    for u in range(_NQ):
        iu = sref[bh * _NQ + u]
        qred[u:u + 1, :] = q_ref[0, 0, pl.ds(iu, 1), :]
    scale = 1.0 / np.sqrt(_F)
    scores = jax.lax.dot_general(qred[...], k_ref[0, 0], (((1,), (1,)), ((), ())),
                                 preferred_element_type=jnp.float32,
                                 precision=_HIGH) * scale
    mx = jnp.max(scores, axis=1, keepdims=True)
    e = jnp.exp(scores - mx)
    attn = e / jnp.sum(e, axis=1, keepdims=True)
    upd = jax.lax.dot_general(attn, v_ref[0, 0], (((1,), (0,)), ((), ())),
                              preferred_element_type=jnp.float32,
                              precision=_HIGH)
    vmean = jnp.mean(v_ref[0, 0], axis=0, keepdims=True)
    vn_ref[0] = jnp.broadcast_to(vmean, (_L, _F))
    for u in range(_NQ):
        iu = sref[bh * _NQ + u]
        vn_ref[0, pl.ds(iu, 1), :] = upd[u:u + 1, :]


def _proj_body(mb_ref, w_ref, bp_ref, o_ref):
    o_ref[0] = jax.lax.dot_general(mb_ref[0], w_ref[...], (((1,), (1,)), ((), ())),
                                   preferred_element_type=jnp.float32,
                                   precision=_HIGH) + bp_ref[...]


def kernel(q, k, v, W_proj, b_proj):
    f32 = jnp.float32
    cnt = jnp.asarray(_CNT_T)

    m = pl.pallas_call(
        _a_body,
        grid=(_BH, _L // _KT),
        in_specs=[
            pl.BlockSpec((1, 1, _L, _F), lambda i, j: (i // _H, i % _H, 0, 0)),
            pl.BlockSpec((1, 1, _KT, _F), lambda i, j: (i // _H, i % _H, j, 0)),
            pl.BlockSpec((_KT, _L), lambda i, j: (j, 0)),
        ],
        out_specs=pl.BlockSpec((1, 1, _L), lambda i, j: (i, 0, 0)),
        out_shape=jax.ShapeDtypeStruct((_BH, 1, _L), f32),
        scratch_shapes=[pltpu.VMEM((1, _L), f32), pltpu.VMEM((1, _L), f32)],
        interpret=_INTERPRET,
    )(q, k, cnt)

    topk = pl.pallas_call(
        _topk_body,
        grid=(1,),
        in_specs=[pl.BlockSpec((_BH, 1, _L), lambda i: (0, 0, 0))],
        out_specs=pl.BlockSpec((_BH, 128), lambda i: (0, 0)),
        out_shape=jax.ShapeDtypeStruct((_BH, 128), jnp.int32),
        interpret=_INTERPRET,
    )(m)

    m_top = topk[:, :_NQ].reshape(-1)

    v_new = pl.pallas_call(
        _attn_body,
        grid_spec=pltpu.PrefetchScalarGridSpec(
            num_scalar_prefetch=1,
            grid=(_BH,),
            in_specs=[
                pl.BlockSpec((1, 1, _L, _F), lambda i, sref: (i // _H, i % _H, 0, 0)),
                pl.BlockSpec((1, 1, _L, _F), lambda i, sref: (i // _H, i % _H, 0, 0)),
                pl.BlockSpec((1, 1, _L, _F), lambda i, sref: (i // _H, i % _H, 0, 0)),
            ],
            out_specs=pl.BlockSpec((1, _L, _F), lambda i, sref: (i, 0, 0)),
            scratch_shapes=[pltpu.VMEM((_NQ, _F), f32)],
        ),
        out_shape=jax.ShapeDtypeStruct((_BH, _L, _F), f32),
        interpret=_INTERPRET,
    )(m_top, q, k, v)

    mb = v_new.reshape(_B, _L, _H * _F)
    bp = b_proj.reshape(1, -1)
    _LT = 512
    out = pl.pallas_call(
        _proj_body,
        grid=(_B, _L // _LT),
        in_specs=[
            pl.BlockSpec((1, _LT, _H * _F), lambda i, j: (i, j, 0)),
            pl.BlockSpec(W_proj.shape, lambda i, j: (0, 0)),
            pl.BlockSpec((1, b_proj.shape[0]), lambda i, j: (0, 0)),
        ],
        out_specs=pl.BlockSpec((1, _LT, b_proj.shape[0]), lambda i, j: (i, j, 0)),
        out_shape=jax.ShapeDtypeStruct((_B, _L, b_proj.shape[0]), f32),
        interpret=_INTERPRET,
    )(mb, W_proj, bp)
    return out


# speedup vs baseline: 1.0690x; 1.0690x over previous
import functools

import numpy as np
import jax
import jax.numpy as jnp
from jax.experimental import pallas as pl
from jax.experimental.pallas import tpu as pltpu

_B, _H, _L, _F = 2, 12, 2048, 64
_BH = _B * _H
_NQ = int(np.ceil(np.log(_L)) * 5)
_NK = _NQ
_KT = 512
_HIGH = jax.lax.Precision.HIGHEST
_DEF = jax.lax.Precision.DEFAULT

_INTERPRET = False


def _rotl32(x, r):
    return ((x << np.uint32(r)) | (x >> np.uint32(32 - r))).astype(np.uint32)


def _threefry2x32_np(ks0, ks1, x0, x1):
    x0 = x0.astype(np.uint32).copy()
    x1 = x1.astype(np.uint32).copy()
    ks2 = np.uint32(np.uint32(ks0) ^ np.uint32(ks1) ^ np.uint32(0x1BD11BDA))
    rots = ((13, 15, 26, 6), (17, 29, 16, 24))
    x0 += np.uint32(ks0)
    x1 += np.uint32(ks1)
    inject = [(ks1, ks2), (ks2, ks0), (ks0, ks1), (ks1, ks2), (ks2, ks0)]
    for d in range(5):
        for r in rots[d % 2]:
            x0 = (x0 + x1).astype(np.uint32)
            x1 = _rotl32(x1, r)
            x1 = x1 ^ x0
        a, b = inject[d]
        x0 = (x0 + np.uint32(a)).astype(np.uint32)
        x1 = (x1 + np.uint32(b) + np.uint32(d + 1)).astype(np.uint32)
    return x0, x1


def _np_randint_key1(shape, span):
    size = int(np.prod(shape))
    flat = np.arange(size, dtype=np.uint64)
    c1 = (flat >> np.uint64(32)).astype(np.uint32)
    c2 = (flat & np.uint64(0xFFFFFFFF)).astype(np.uint32)
    sb1, sb2 = _threefry2x32_np(np.uint32(0), np.uint32(1),
                                np.zeros(2, np.uint32), np.arange(2, dtype=np.uint32))
    h1, h2 = _threefry2x32_np(sb1[0], sb2[0], c1, c2)
    l1, l2 = _threefry2x32_np(sb1[1], sb2[1], c1, c2)
    higher = (h1 ^ h2).reshape(shape)
    lower = (l1 ^ l2).reshape(shape)
    spanu = np.uint32(span)
    mult = np.uint32((int(2 ** 16) % int(span)) ** 2 % int(span))
    off = ((higher % spanu) * mult + (lower % spanu)) % spanu
    return off.astype(np.int32)


def _build_cnt():
    try:
        cpu = jax.devices("cpu")[0]
        with jax.default_device(cpu):
            idx = np.asarray(jax.random.randint(jax.random.key(1), (_L, _NK), 0, _L))
    except Exception:
        idx = _np_randint_key1((_L, _NK), _L)
    cnt = np.zeros((_L, _L), np.float32)
    np.add.at(cnt, (np.arange(_L)[:, None], idx), 1.0)
    cnt_t = np.ascontiguousarray(cnt.T)
    bias_t = np.where(cnt_t > 0, np.float32(0.0), np.float32(-1e30)).astype(np.float32)
    return cnt_t, bias_t


_CNT_T, _BIAS_T = _build_cnt()


def _a_body(q_ref, k_ref, cnt_ref, bias_ref, m_ref, smax, ssum):
    kt = pl.program_id(0)
    j = pl.program_id(1)
    qh = q_ref[0, 0]
    _HK = _KT // 2
    sts, pmaxs, psums = [], [], []
    for c in range(2):
        kc = k_ref[0, 0, c * _HK:(c + 1) * _HK, :]
        st = jax.lax.dot_general(kc, qh, (((1,), (1,)), ((), ())),
                                 preferred_element_type=jnp.float32,
                                 precision=_HIGH)
        pmaxs.append(jnp.max(st + bias_ref[c * _HK:(c + 1) * _HK, :],
                             axis=0, keepdims=True))
        psums.append(jnp.sum(cnt_ref[c * _HK:(c + 1) * _HK, :] * st,
                             axis=0, keepdims=True))
    pmax = jnp.maximum(pmaxs[0], pmaxs[1])
    psum = psums[0] + psums[1]
    row = pl.ds(j, 1)

    @pl.when(kt == 0)
    def _():
        smax[row, :] = pmax
        ssum[row, :] = psum

    @pl.when(kt > 0)
    def _():
        smax[row, :] = jnp.maximum(smax[row, :], pmax)
        ssum[row, :] = ssum[row, :] + psum

    @pl.when(kt == (_L // _KT) - 1)
    def _():
        m_ref[0] = smax[row, :] - ssum[row, :] * (1.0 / _L)


def _topk_body(m_ref, idx_ref):
    mv = m_ref[:, 0, :]
    lidx = jax.lax.broadcasted_iota(jnp.int32, (_BH, _L), 1)
    lane = jax.lax.broadcasted_iota(jnp.int32, (_BH, 128), 1)
    acc = jnp.zeros((_BH, 128), jnp.int32)
    for i in range(_NQ):
        mx = jnp.max(mv, axis=1, keepdims=True)
        il = jnp.min(jnp.where(mv == mx, lidx, _L), axis=1, keepdims=True)
        acc = acc + jnp.where(lane == i, jnp.broadcast_to(il, (_BH, 128)), 0)
        mv = jnp.where(lidx == il, -jnp.inf, mv)
    idx_ref[...] = acc


def _attn_body(sref, q_ref, k_ref, v_ref, vn_ref, qred):
    bh = pl.program_id(0)
    for u in range(_NQ):
        iu = sref[bh * _NQ + u]
        qred[u:u + 1, :] = q_ref[0, 0, pl.ds(iu, 1), :]
    scale = 1.0 / np.sqrt(_F)
    scores = jax.lax.dot_general(qred[...], k_ref[0, 0], (((1,), (1,)), ((), ())),
                                 preferred_element_type=jnp.float32,
                                 precision=_HIGH) * scale
    mx = jnp.max(scores, axis=1, keepdims=True)
    e = jnp.exp(scores - mx)
    attn = e / jnp.sum(e, axis=1, keepdims=True)
    upd = jax.lax.dot_general(attn, v_ref[0, 0], (((1,), (0,)), ((), ())),
                              preferred_element_type=jnp.float32,
                              precision=_HIGH)
    vmean = jnp.mean(v_ref[0, 0], axis=0, keepdims=True)
    vn_ref[0] = jnp.broadcast_to(vmean, (_L, _F))
    for u in range(_NQ):
        iu = sref[bh * _NQ + u]
        vn_ref[0, pl.ds(iu, 1), :] = upd[u:u + 1, :]


def _proj_body(mb_ref, w_ref, bp_ref, o_ref):
    o_ref[0] = jax.lax.dot_general(mb_ref[0], w_ref[...], (((1,), (1,)), ((), ())),
                                   preferred_element_type=jnp.float32,
                                   precision=_HIGH) + bp_ref[...]


def kernel(q, k, v, W_proj, b_proj):
    f32 = jnp.float32
    cnt = jnp.asarray(_CNT_T)
    bias = jnp.asarray(_BIAS_T)

    m = pl.pallas_call(
        _a_body,
        grid=(_L // _KT, _BH),
        in_specs=[
            pl.BlockSpec((1, 1, _L, _F), lambda t, j: (j // _H, j % _H, 0, 0)),
            pl.BlockSpec((1, 1, _KT, _F), lambda t, j: (j // _H, j % _H, t, 0)),
            pl.BlockSpec((_KT, _L), lambda t, j: (t, 0)),
            pl.BlockSpec((_KT, _L), lambda t, j: (t, 0)),
        ],
        out_specs=pl.BlockSpec((1, 1, _L), lambda t, j: (j, 0, 0)),
        out_shape=jax.ShapeDtypeStruct((_BH, 1, _L), f32),
        scratch_shapes=[pltpu.VMEM((_BH, _L), f32), pltpu.VMEM((_BH, _L), f32)],
        interpret=_INTERPRET,
    )(q, k, cnt, bias)

    topk = pl.pallas_call(
        _topk_body,
        grid=(1,),
        in_specs=[pl.BlockSpec((_BH, 1, _L), lambda i: (0, 0, 0))],
        out_specs=pl.BlockSpec((_BH, 128), lambda i: (0, 0)),
        out_shape=jax.ShapeDtypeStruct((_BH, 128), jnp.int32),
        interpret=_INTERPRET,
    )(m)

    m_top = topk[:, :_NQ].reshape(-1)

    v_new = pl.pallas_call(
        _attn_body,
        grid_spec=pltpu.PrefetchScalarGridSpec(
            num_scalar_prefetch=1,
            grid=(_BH,),
            in_specs=[
                pl.BlockSpec((1, 1, _L, _F), lambda i, sref: (i // _H, i % _H, 0, 0)),
                pl.BlockSpec((1, 1, _L, _F), lambda i, sref: (i // _H, i % _H, 0, 0)),
                pl.BlockSpec((1, 1, _L, _F), lambda i, sref: (i // _H, i % _H, 0, 0)),
            ],
            out_specs=pl.BlockSpec((1, _L, _F), lambda i, sref: (i, 0, 0)),
            scratch_shapes=[pltpu.VMEM((_NQ, _F), f32)],
        ),
        out_shape=jax.ShapeDtypeStruct((_BH, _L, _F), f32),
        interpret=_INTERPRET,
    )(m_top, q, k, v)

    mb = v_new.reshape(_B, _L, _H * _F)
    bp = b_proj.reshape(1, -1)
    _LT = 512
    out = pl.pallas_call(
        _proj_body,
        grid=(_B, _L // _LT),
        in_specs=[
            pl.BlockSpec((1, _LT, _H * _F), lambda i, j: (i, j, 0)),
            pl.BlockSpec(W_proj.shape, lambda i, j: (0, 0)),
            pl.BlockSpec((1, b_proj.shape[0]), lambda i, j: (0, 0)),
        ],
        out_specs=pl.BlockSpec((1, _LT, b_proj.shape[0]), lambda i, j: (i, j, 0)),
        out_shape=jax.ShapeDtypeStruct((_B, _L, b_proj.shape[0]), f32),
        interpret=_INTERPRET,
    )(mb, W_proj, bp)
    return out
